# baseline (speedup 1.0000x reference)
import functools
import jax
import jax.numpy as jnp
from jax import lax
from jax.experimental import pallas as pl
from jax.experimental.pallas import tpu as pltpu
from jax.experimental.pallas import tpu_sc as plsc


def kernel(inputs, table):
    batch, seq = inputs.shape
    n_rows, d_model = table.shape
    n_chunks_per_b = 32 // batch
    chunk = seq // n_chunks_per_b

    mesh = plsc.VectorSubcoreMesh(core_axis_name="c", subcore_axis_name="s")

    @functools.partial(
        pl.kernel,
        mesh=mesh,
        out_type=jax.ShapeDtypeStruct((batch, seq, d_model), table.dtype),
        scratch_types=[pltpu.SemaphoreType.DMA],
    )
    def sc_copy(table_hbm, out_hbm, sem):
        c = lax.axis_index("c")
        s = lax.axis_index("s")
        wid = s * 2 + c
        b = wid // n_chunks_per_b
        off = (wid % n_chunks_per_b) * chunk
        pltpu.async_copy(
            table_hbm.at[pl.ds(off, chunk), :],
            out_hbm.at[b, pl.ds(off, chunk), :],
            sem,
        ).wait()

    return sc_copy(table)


# speedup vs baseline: 33.3147x; 33.3147x over previous
import functools
import jax
import jax.numpy as jnp
from jax import lax
from jax.experimental import pallas as pl
from jax.experimental.pallas import tpu as pltpu
from jax.experimental.pallas import tpu_sc as plsc

PIECE = 64


def kernel(inputs, table):
    batch, seq = inputs.shape
    n_rows, d_model = table.shape
    n_chunks_per_b = 32 // batch
    chunk = seq // n_chunks_per_b
    n_pieces = chunk // PIECE

    mesh = plsc.VectorSubcoreMesh(core_axis_name="c", subcore_axis_name="s")

    @functools.partial(
        pl.kernel,
        mesh=mesh,
        out_type=jax.ShapeDtypeStruct((batch, seq, d_model), table.dtype),
        scratch_types=[
            pltpu.VMEM((2, PIECE, d_model), table.dtype),
            pltpu.SemaphoreType.DMA((2,)),
            pltpu.SemaphoreType.DMA((2,)),
        ],
    )
    def sc_copy(table_hbm, out_hbm, buf, rsem, wsem):
        c = lax.axis_index("c")
        s = lax.axis_index("s")
        wid = s * 2 + c
        b = wid // n_chunks_per_b
        off = (wid % n_chunks_per_b) * chunk

        def rd(k):
            return pltpu.make_async_copy(
                table_hbm.at[pl.ds(off + k * PIECE, PIECE), :],
                buf.at[k % 2],
                rsem.at[k % 2],
            )

        def wr(k):
            return pltpu.make_async_copy(
                buf.at[k % 2],
                out_hbm.at[b, pl.ds(off + k * PIECE, PIECE), :],
                wsem.at[k % 2],
            )

        rd(0).start()
        for k in range(n_pieces):
            if k + 1 < n_pieces:
                if k >= 1:
                    wr(k - 1).wait()
                rd(k + 1).start()
            rd(k).wait()
            wr(k).start()
        wr(n_pieces - 2).wait()
        wr(n_pieces - 1).wait()

    return sc_copy(table)


# speedup vs baseline: 51.0029x; 1.5309x over previous
import functools
import jax
import jax.numpy as jnp
from jax import lax
from jax.experimental import pallas as pl
from jax.experimental.pallas import tpu as pltpu
from jax.experimental.pallas import tpu_sc as plsc

PIECE = 64
NWORK = 32


def kernel(inputs, table):
    batch, seq = inputs.shape
    n_rows, d_model = table.shape
    chunk = seq // NWORK
    n_pieces = chunk // PIECE

    mesh = plsc.VectorSubcoreMesh(core_axis_name="c", subcore_axis_name="s")

    @functools.partial(
        pl.kernel,
        mesh=mesh,
        out_type=jax.ShapeDtypeStruct((batch, seq, d_model), table.dtype),
        scratch_types=[
            pltpu.VMEM((2, PIECE, d_model), table.dtype),
            pltpu.SemaphoreType.DMA((2,)),
            pltpu.SemaphoreType.DMA((2, 4)),
        ],
    )
    def sc_copy(table_hbm, out_hbm, buf, rsem, wsem):
        c = lax.axis_index("c")
        s = lax.axis_index("s")
        wid = s * 2 + c
        off = wid * chunk

        def rd(k):
            return pltpu.make_async_copy(
                table_hbm.at[pl.ds(off + k * PIECE, PIECE), :],
                buf.at[k % 2],
                rsem.at[k % 2],
            )

        def wr(k, b):
            return pltpu.make_async_copy(
                buf.at[k % 2],
                out_hbm.at[b, pl.ds(off + k * PIECE, PIECE), :],
                wsem.at[k % 2, b],
            )

        rd(0).start()
        for k in range(n_pieces):
            if k + 1 < n_pieces:
                if k >= 1:
                    for b in range(batch):
                        wr(k - 1, b).wait()
                rd(k + 1).start()
            rd(k).wait()
            for b in range(batch):
                wr(k, b).start()
        for k in (n_pieces - 2, n_pieces - 1):
            for b in range(batch):
                wr(k, b).wait()

    return sc_copy(table)
